# baseline (speedup 1.0000x reference)
import functools

import jax
import jax.numpy as jnp
from jax import lax
from jax.experimental import pallas as pl
from jax.experimental.pallas import tpu as pltpu
from jax.experimental.pallas import tpu_sc as plsc

B = 1024
L = 200
V = 100000
D = 64

NC = 2
NS = 16
NW = NC * NS
NBL = B * L
PW = NBL // NW
C = 128
NCH = PW // C

def _splat(v, k):
  dn = lax.GatherDimensionNumbers(
      offset_dims=(), collapsed_slice_dims=(0,), start_index_map=(0,))
  idx = jnp.full((16, 1), k, dtype=jnp.int32)
  return lax.gather(v, idx, dn, slice_sizes=(1,),
                    mode=lax.GatherScatterMode.PROMISE_IN_BOUNDS)


def _sc_body(tok_hbm, typ_hbm, ttab_hbm, seg_hbm, pos_hbm, out_hbm,
             idx_v, typ_v, seg_v, pos_v, pose_v, rows_v, gsem):
  wid = lax.axis_index("c") * NS + lax.axis_index("s")

  pltpu.sync_copy(tok_hbm.at[wid], idx_v)
  pltpu.sync_copy(typ_hbm.at[wid], typ_v)
  pltpu.sync_copy(seg_hbm, seg_v)
  pltpu.sync_copy(pos_hbm, pos_v)

  def build(l, carry):
    for j in range(D // 16):
      sl = pl.ds(j * 16, 16)
      p = pos_v[l, sl] + seg_v[0, sl]
      pose_v[l, sl] = p
      pose_v[L + l, sl] = p
    return carry
  lax.fori_loop(0, L, build, 0)

  dsegs = tuple(seg_v[1, pl.ds(j * 16, 16)] - seg_v[0, pl.ds(j * 16, 16)]
                for j in range(D // 16))

  def chunk(c, carry):
    base = wid * PW + c * C
    lbase = lax.rem(c * C, L)
    pltpu.async_copy(ttab_hbm.at[idx_v.at[c]], rows_v, gsem).wait()

    def grp(g, gcarry):
      dsg = gcarry
      r0 = g * 16
      tvf = typ_v[pl.ds(c * C + r0, 16)].astype(jnp.float32)
      for k in range(16):
        r = r0 + k
        tb = _splat(tvf, k)
        for j in range(D // 16):
          sl = pl.ds(j * 16, 16)
          rows_v[r, sl] = (rows_v[r, sl] + pose_v[lbase + r, sl]
                           + tb * dsg[j])
      return gcarry
    lax.fori_loop(0, C // 16, grp, dsegs)

    pltpu.sync_copy(rows_v, out_hbm.at[pl.ds(base, C)])
    return carry
  lax.fori_loop(0, NCH, chunk, 0)


@jax.jit
def _sc_embed(tok3, typ2, ttab, seg, pos):
  return pl.kernel(
      _sc_body,
      out_type=jax.ShapeDtypeStruct((NBL, D), jnp.float32),
      mesh=plsc.VectorSubcoreMesh(core_axis_name="c", subcore_axis_name="s"),
      compiler_params=pltpu.CompilerParams(use_tc_tiling_on_sc=False),
      scratch_types=[
          pltpu.VMEM((NCH, C), jnp.int32),
          pltpu.VMEM((PW,), jnp.int32),
          pltpu.VMEM((2, D), jnp.float32),
          pltpu.VMEM((L, D), jnp.float32),
          pltpu.VMEM((2 * L, D), jnp.float32),
          pltpu.VMEM((C, D), jnp.float32),
          pltpu.SemaphoreType.DMA,
      ],
  )(tok3, typ2, ttab, seg, pos)


def _mask_body(ids_ref, out_ref):
  out_ref[...] = ids_ref[...] != 0


@jax.jit
def _mask_call(token_ids):
  return pl.pallas_call(
      _mask_body,
      out_shape=jax.ShapeDtypeStruct((B, L), jnp.bool_),
  )(token_ids)


def kernel(token_ids, type_token_ids, token_table, segment_table, positional_table):
  tok3 = token_ids.astype(jnp.int32).reshape(NW, NCH, C)
  typ2 = type_token_ids.astype(jnp.int32).reshape(NW, PW)
  out = _sc_embed(tok3, typ2, token_table, segment_table, positional_table)
  outputs = out.reshape(B, L, D)
  attention_mask = _mask_call(token_ids).reshape(B, 1, 1, L)
  return outputs, attention_mask


# speedup vs baseline: 1.2400x; 1.2400x over previous
import functools

import jax
import jax.numpy as jnp
from jax import lax
from jax.experimental import pallas as pl
from jax.experimental.pallas import tpu as pltpu
from jax.experimental.pallas import tpu_sc as plsc

B = 1024
L = 200
V = 100000
D = 64

NC = 2
NS = 16
NW = NC * NS
NBL = B * L
PW = NBL // NW
C = 128
NCH = PW // C

def _sc_body(tok_hbm, civ_hbm, ttab_hbm, comb_hbm, out_hbm,
             idx_v, civ_v, rows_v, add_v, gs1, gs2):
  wid = lax.axis_index("c") * NS + lax.axis_index("s")

  pltpu.sync_copy(tok_hbm.at[wid], idx_v)
  pltpu.sync_copy(civ_hbm.at[wid], civ_v)

  def chunk(c, carry):
    base = wid * PW + c * C
    cp1 = pltpu.async_copy(ttab_hbm.at[idx_v.at[c]], rows_v, gs1)
    cp2 = pltpu.async_copy(comb_hbm.at[civ_v.at[c]], add_v, gs2)
    cp1.wait()
    cp2.wait()

    def row(r, rc):
      for j in range(D // 16):
        sl = pl.ds(j * 16, 16)
        plsc.addupdate(rows_v.at[r, sl], add_v[r, sl])
      return rc
    lax.fori_loop(0, C, row, 0)

    pltpu.sync_copy(rows_v, out_hbm.at[pl.ds(base, C)])
    return carry
  lax.fori_loop(0, NCH, chunk, 0)


@jax.jit
def _sc_embed(tok3, civ3, ttab, comb):
  return pl.kernel(
      _sc_body,
      out_type=jax.ShapeDtypeStruct((NBL, D), jnp.float32),
      mesh=plsc.VectorSubcoreMesh(core_axis_name="c", subcore_axis_name="s"),
      compiler_params=pltpu.CompilerParams(use_tc_tiling_on_sc=False),
      scratch_types=[
          pltpu.VMEM((NCH, C), jnp.int32),
          pltpu.VMEM((NCH, C), jnp.int32),
          pltpu.VMEM((C, D), jnp.float32),
          pltpu.VMEM((C, D), jnp.float32),
          pltpu.SemaphoreType.DMA,
          pltpu.SemaphoreType.DMA,
      ],
  )(tok3, civ3, ttab, comb)


def _prep_body(ids_ref, typ_ref, seg_ref, pos_ref, mask_ref, comb_ref, civ_ref):
  mask_ref[...] = ids_ref[...] != 0
  l_iota = lax.broadcasted_iota(jnp.int32, (B, L), 1)
  civ_ref[...] = typ_ref[...] * L + l_iota
  seg = seg_ref[...]
  pos = pos_ref[...]
  comb_ref[...] = jnp.concatenate([pos + seg[0:1, :], pos + seg[1:2, :]],
                                  axis=0)


@jax.jit
def _prep_call(token_ids, type_token_ids, segment_table, positional_table):
  return pl.pallas_call(
      _prep_body,
      out_shape=(
          jax.ShapeDtypeStruct((B, L), jnp.bool_),
          jax.ShapeDtypeStruct((2 * L, D), jnp.float32),
          jax.ShapeDtypeStruct((B, L), jnp.int32),
      ),
  )(token_ids, type_token_ids, segment_table, positional_table)


def kernel(token_ids, type_token_ids, token_table, segment_table, positional_table):
  token_ids = token_ids.astype(jnp.int32)
  type_token_ids = type_token_ids.astype(jnp.int32)
  mask, comb, civ = _prep_call(token_ids, type_token_ids, segment_table,
                               positional_table)
  tok3 = token_ids.reshape(NW, NCH, C)
  civ3 = civ.reshape(NW, NCH, C)
  out = _sc_embed(tok3, civ3, token_table, comb)
  outputs = out.reshape(B, L, D)
  attention_mask = mask.reshape(B, 1, 1, L)
  return outputs, attention_mask
